# baseline (speedup 1.0000x reference)
import jax
import jax.numpy as jnp
from jax import lax
from jax.experimental import pallas as pl
from jax.experimental.pallas import tpu as pltpu
from jax.experimental.pallas import tpu_sc as plsc

N_NODES = 10000
N_EDGES = 320000
D = 128
HEADS = 4

NPAD = 10240
NC, NS, LANES = 2, 16, 16
E_FULL = N_EDGES + N_NODES
EB = 128
BATCHES = 162
EPT = EB * BATCHES
EPAD = EPT * NS
HALF = NPAD // NC
ACC_ROWS = HALF + EB
ROWS_PER_TILE = ACC_ROWS // NS
DEN_R = NPAD // LANES


def _sc_att_body(xl_hbm, xr_hbm, att_hbm, src_hbm, dst_hbm,
                 feat_hbm, den_hbm,
                 src_v, dst_v, loc_idx, xl_rows, xr_rows, out_rows, att_v,
                 den_tile, shared, sem):
    c = lax.axis_index("c")
    s = lax.axis_index("s")
    lane = lax.iota(jnp.int32, LANES)

    pltpu.sync_copy(att_hbm, att_v)
    att_c = [att_v[pl.ds(k * LANES, LANES)] for k in range(D // LANES)]

    def _zero_rows(r, carry):
        for kk in range(D // LANES):
            out_rows[r, pl.ds(kk * LANES, LANES)] = jnp.zeros((LANES,), jnp.float32)
        return carry

    lax.fori_loop(0, EB, _zero_rows, 0)

    def _zero_den(r, carry):
        den_tile[pl.ds(r * LANES, LANES)] = jnp.zeros((LANES,), jnp.float32)
        return carry

    lax.fori_loop(0, DEN_R, _zero_den, 0)

    for off, nr in ((0, EB), (EB, EB), (2 * EB, ROWS_PER_TILE - 2 * EB)):
        pltpu.sync_copy(
            out_rows.at[pl.ds(0, nr)],
            shared.at[pl.ds(s * ROWS_PER_TILE + off, nr)])
    plsc.subcore_barrier()

    def _batch(b, carry):
        base = s * EPT + b * EB
        pltpu.sync_copy(src_hbm.at[pl.ds(base, EB)], src_v)
        pltpu.sync_copy(dst_hbm.at[pl.ds(base, EB)], dst_v)
        cp1 = pltpu.async_copy(xl_hbm.at[src_v], xl_rows, sem)
        cp2 = pltpu.async_copy(xr_hbm.at[dst_v], xr_rows, sem)
        for g in range(EB // LANES):
            dvg = dst_v[pl.ds(g * LANES, LANES)]
            loc = dvg - c * HALF
            ok = (loc >= 0) & (loc < HALF)
            loc_idx[pl.ds(g * LANES, LANES)] = jnp.where(
                ok, loc, jnp.full((LANES,), HALF, jnp.int32))
        cp1.wait()
        cp2.wait()

        for g in range(EB // LANES):
            dvg = dst_v[pl.ds(g * LANES, LANES)]

            def _edge(j, pacc):
                e = g * LANES + j
                xs = []
                acc = jnp.zeros((LANES,), jnp.float32)
                for k in range(D // LANES):
                    xlk = xl_rows[e, pl.ds(k * LANES, LANES)]
                    xs.append(xlk)
                    t = xlk + xr_rows[e, pl.ds(k * LANES, LANES)]
                    t = jnp.maximum(t, 0.2 * t)
                    acc = acc + t * att_c[k]
                for step in (8, 4, 2, 1):
                    acc = acc + acc.at[lane ^ step].get(mode="promise_in_bounds")
                pvec = jnp.exp(acc)
                for k in range(D // LANES):
                    out_rows[e, pl.ds(k * LANES, LANES)] = xs[k] * pvec
                return jnp.where(lane == j, pvec, pacc)

            pacc = lax.fori_loop(0, LANES, _edge,
                                 jnp.zeros((LANES,), jnp.float32))
            plsc.addupdate_scatter(den_tile, [dvg], pacc)

        pltpu.sync_copy(out_rows, shared.at[loc_idx], add=True)
        return carry

    lax.fori_loop(0, BATCHES, _batch, 0)
    plsc.subcore_barrier()

    pltpu.sync_copy(shared.at[pl.ds(s * ROWS_PER_TILE, ROWS_PER_TILE)],
                    feat_hbm.at[c, pl.ds(s * ROWS_PER_TILE, ROWS_PER_TILE)])
    pltpu.sync_copy(den_tile, den_hbm.at[c, s])


@jax.jit
def _sc_att(xl, xr, att, src, dst):
    mesh = plsc.VectorSubcoreMesh(core_axis_name="c", subcore_axis_name="s")
    return pl.kernel(
        _sc_att_body,
        out_type=(
            jax.ShapeDtypeStruct((NC, ACC_ROWS, D), jnp.float32),
            jax.ShapeDtypeStruct((NC, NS, NPAD), jnp.float32),
        ),
        mesh=mesh,
        compiler_params=pltpu.CompilerParams(needs_layout_passes=False),
        scratch_types=[
            pltpu.VMEM((EB,), jnp.int32),
            pltpu.VMEM((EB,), jnp.int32),
            pltpu.VMEM((EB,), jnp.int32),
            pltpu.VMEM((EB, D), jnp.float32),
            pltpu.VMEM((EB, D), jnp.float32),
            pltpu.VMEM((EB, D), jnp.float32),
            pltpu.VMEM((D,), jnp.float32),
            pltpu.VMEM((NPAD,), jnp.float32),
            pltpu.VMEM_SHARED((ACC_ROWS, D), jnp.float32),
            pltpu.SemaphoreType.DMA,
        ],
    )(xl, xr, att, src, dst)


_RT = 512
_NRT = NPAD // _RT
_HRT = HALF // _RT
NSC = NC * NS


def _mm1_body(x_ref, w_ref, o_ref):
    o_ref[0] = jnp.dot(x_ref[...], w_ref[...], preferred_element_type=jnp.float32)


@jax.jit
def _mm1(xp, wcat):
    return pl.pallas_call(
        _mm1_body,
        grid=(2 * HEADS, _NRT),
        in_specs=[
            pl.BlockSpec((_RT, D), lambda j, i: (i, 0)),
            pl.BlockSpec((D, D), lambda j, i: (0, j)),
        ],
        out_specs=pl.BlockSpec((1, _RT, D), lambda j, i: (j, i, 0)),
        out_shape=jax.ShapeDtypeStruct((2 * HEADS, NPAD, D), jnp.float32),
    )(xp, wcat)


def _norm_head(feat_ref, den_ref):
    num = feat_ref[0]
    den = (0.5 * jnp.sum(den_ref[...], axis=0))[:, None]
    return num / (den + 1e-16)


def _mid_body(f0, f1, f2, f3, d0, d1, d2, d3, b1_ref, w_ref, o_ref):
    hs = [_norm_head(f, d) for f, d in ((f0, d0), (f1, d1), (f2, d2), (f3, d3))]
    h = jnp.concatenate(hs, axis=1) + b1_ref[0]
    h = jnp.where(h > 0.0, h, jnp.exp(jnp.minimum(h, 0.0)) - 1.0)
    o_ref[0] = jnp.dot(h, w_ref[...], preferred_element_type=jnp.float32)


@jax.jit
def _mid(p0, p1, p2, p3, b1r, wcat2):
    fspec = pl.BlockSpec((1, _RT, D), lambda j, i: (i // _HRT, i % _HRT, 0))
    dspec = pl.BlockSpec((NSC, _RT), lambda j, i: (0, i))
    return pl.pallas_call(
        _mid_body,
        grid=(2, _NRT),
        in_specs=[fspec, fspec, fspec, fspec, dspec, dspec, dspec, dspec,
                  pl.BlockSpec((1, HEADS * D), lambda j, i: (0, 0)),
                  pl.BlockSpec((HEADS * D, D), lambda j, i: (0, j))],
        out_specs=pl.BlockSpec((1, _RT, D), lambda j, i: (j, i, 0)),
        out_shape=jax.ShapeDtypeStruct((2, NPAD, D), jnp.float32),
    )(p0[0], p1[0], p2[0], p3[0],
      p0[1].reshape(NSC, NPAD), p1[1].reshape(NSC, NPAD),
      p2[1].reshape(NSC, NPAD), p3[1].reshape(NSC, NPAD), b1r, wcat2)


def _fin_body(f_ref, d_ref, b2_ref, o_ref):
    o_ref[...] = _norm_head(f_ref, d_ref) + b2_ref[0]


@jax.jit
def _fin(q, b2r):
    return pl.pallas_call(
        _fin_body,
        grid=(_NRT,),
        in_specs=[
            pl.BlockSpec((1, _RT, D), lambda i: (i // _HRT, i % _HRT, 0)),
            pl.BlockSpec((NSC, _RT), lambda i: (0, i)),
            pl.BlockSpec((1, D), lambda i: (0, 0)),
        ],
        out_specs=pl.BlockSpec((_RT, D), lambda i: (i, 0)),
        out_shape=jax.ShapeDtypeStruct((NPAD, D), jnp.float32),
    )(q[0], q[1].reshape(NSC, NPAD), b2r)


def kernel(x, edge_index, W1l, W1r, a1, b1, W2l, W2r, a2, b2):
    xp = jnp.zeros((NPAD, D), jnp.float32).at[:N_NODES].set(x)
    loop = jnp.arange(N_NODES, dtype=jnp.int32)
    pad = jnp.full((EPAD - E_FULL,), N_NODES, dtype=jnp.int32)
    src = jnp.concatenate([edge_index[0].astype(jnp.int32), loop, pad])
    dst = jnp.concatenate([edge_index[1].astype(jnp.int32), loop, pad])

    wcat1 = jnp.concatenate([W1l, W1r], axis=1)
    y1 = _mm1(xp, wcat1)

    parts = [
        _sc_att(y1[h], y1[HEADS + h], a1[h], src, dst) for h in range(HEADS)
    ]

    wcat2 = jnp.concatenate([W2l, W2r], axis=1)
    y2 = _mid(parts[0], parts[1], parts[2], parts[3],
              b1.reshape(1, HEADS * D), wcat2)

    q = _sc_att(y2[0], y2[1], a2[0], src, dst)
    out = _fin(q, b2.reshape(1, D))
    return out[:N_NODES]


# speedup vs baseline: 1.5281x; 1.5281x over previous
import jax
import jax.numpy as jnp
from jax import lax
from jax.experimental import pallas as pl
from jax.experimental.pallas import tpu as pltpu
from jax.experimental.pallas import tpu_sc as plsc

N_NODES = 10000
N_EDGES = 320000
D = 128
HEADS = 4

NPAD = 10240
NC, NS, LANES = 2, 16, 16
E_FULL = N_EDGES + N_NODES
EB = 96
BATCHES = 216
PAIRS = BATCHES // 2
EPT = EB * BATCHES
EPAD = EPT * NS
HALF = NPAD // NC
ACC_ROWS = HALF + 128
ROWS_PER_TILE = ACC_ROWS // NS

NDG = NPAD // D
DG_SLAB = 8
DG_TILES = NDG // DG_SLAB


def _sc_att_body(xl_hbm, xr_hbm, att_hbm, src_hbm, dst_hbm,
                 feat_hbm, den_hbm,
                 src_a, src_b, dst_a, dst_b, den_idx, loc_idx,
                 xl_a, xr_a, xl_b, xr_b,
                 out_rows, den_rows, att_v, shared, den_acc, sem_a, sem_b):
    c = lax.axis_index("c")
    s = lax.axis_index("s")
    lane = lax.iota(jnp.int32, LANES)

    pltpu.sync_copy(att_hbm, att_v)
    dst_a[pl.ds(EB, LANES)] = jnp.zeros((LANES,), jnp.int32)
    dst_b[pl.ds(EB, LANES)] = jnp.zeros((LANES,), jnp.int32)
    att_c = [att_v[pl.ds(k * LANES, LANES)] for k in range(D // LANES)]
    lanefull = [lane + (k * LANES) for k in range(D // LANES)]

    def _zero_rows(r, carry):
        for kk in range(D // LANES):
            out_rows[r, pl.ds(kk * LANES, LANES)] = jnp.zeros((LANES,), jnp.float32)
        return carry

    lax.fori_loop(0, EB, _zero_rows, 0)

    for off, nr in ((0, EB), (EB, EB), (2 * EB, EB),
                    (3 * EB, ROWS_PER_TILE - 3 * EB)):
        pltpu.sync_copy(
            out_rows.at[pl.ds(0, nr)],
            shared.at[pl.ds(s * ROWS_PER_TILE + off, nr)])

    @pl.when(s < DG_TILES)
    def _zero_den():
        pltpu.sync_copy(out_rows.at[pl.ds(0, DG_SLAB)],
                        den_acc.at[pl.ds(s * DG_SLAB, DG_SLAB)])

    plsc.subcore_barrier()

    def _issue(base, src_v, dst_v, xl_rows, xr_rows, sem):
        pltpu.sync_copy(src_hbm.at[pl.ds(base, EB)], src_v)
        pltpu.sync_copy(dst_hbm.at[pl.ds(base, EB)], dst_v.at[pl.ds(0, EB)])
        pltpu.async_copy(xl_hbm.at[src_v], xl_rows, sem)
        pltpu.async_copy(xr_hbm.at[dst_v.at[pl.ds(0, EB)]], xr_rows, sem)

    def _wait(src_v, dst_v, xl_rows, xr_rows, sem):
        pltpu.make_async_copy(xl_hbm.at[src_v], xl_rows, sem).wait()
        pltpu.make_async_copy(xr_hbm.at[dst_v.at[pl.ds(0, EB)]], xr_rows, sem).wait()

    def _process(dst_v, xl_rows, xr_rows):
        for g in range(EB // LANES):
            dvg = dst_v[pl.ds(g * LANES, LANES)]
            den_idx[pl.ds(g * LANES, LANES)] = lax.shift_right_logical(dvg, 7)
            loc = dvg - c * HALF
            ok = (loc >= 0) & (loc < HALF)
            loc_idx[pl.ds(g * LANES, LANES)] = jnp.where(
                ok, loc, jnp.full((LANES,), HALF, jnp.int32))

        def _edge(e, ecarry):
            xs = []
            acc = jnp.zeros((LANES,), jnp.float32)
            for k in range(D // LANES):
                xlk = xl_rows[e, pl.ds(k * LANES, LANES)]
                xs.append(xlk)
                t = xlk + xr_rows[e, pl.ds(k * LANES, LANES)]
                t = jnp.where(t >= 0.0, t, 0.2 * t)
                acc = acc + t * att_c[k]
            for step in (8, 4, 2, 1):
                acc = acc + acc.at[lane ^ step].get(mode="promise_in_bounds")
            pvec = jnp.exp(acc)
            dv = dst_v[pl.ds(e, LANES)]
            dm = jnp.full((LANES,), dv[0] & (D - 1), jnp.int32)
            for k in range(D // LANES):
                out_rows[e, pl.ds(k * LANES, LANES)] = xs[k] * pvec
                den_rows[e, pl.ds(k * LANES, LANES)] = jnp.where(
                    lanefull[k] == dm, pvec, jnp.zeros((LANES,), jnp.float32))
            return ecarry

        lax.fori_loop(0, EB, _edge, 0)
        pltpu.sync_copy(out_rows, shared.at[loc_idx], add=True)
        pltpu.sync_copy(den_rows, den_acc.at[den_idx], add=True)

    base0 = s * EPT
    _issue(base0, src_a, dst_a, xl_a, xr_a, sem_a)

    def _pair(i, carry):
        base_a = s * EPT + (2 * i) * EB
        base_b = base_a + EB
        base_n = s * EPT + jnp.minimum(2 * i + 2, BATCHES - 1) * EB
        _wait(src_a, dst_a, xl_a, xr_a, sem_a)
        _issue(base_b, src_b, dst_b, xl_b, xr_b, sem_b)
        _process(dst_a, xl_a, xr_a)
        _wait(src_b, dst_b, xl_b, xr_b, sem_b)
        _issue(base_n, src_a, dst_a, xl_a, xr_a, sem_a)
        _process(dst_b, xl_b, xr_b)
        return carry

    lax.fori_loop(0, PAIRS, _pair, 0)
    _wait(src_a, dst_a, xl_a, xr_a, sem_a)
    plsc.subcore_barrier()

    pltpu.sync_copy(shared.at[pl.ds(s * ROWS_PER_TILE, ROWS_PER_TILE)],
                    feat_hbm.at[c, pl.ds(s * ROWS_PER_TILE, ROWS_PER_TILE)])

    @pl.when(s < DG_TILES)
    def _dump_den():
        pltpu.sync_copy(den_acc.at[pl.ds(s * DG_SLAB, DG_SLAB)],
                        den_hbm.at[c, pl.ds(s * DG_SLAB, DG_SLAB)])


@jax.jit
def _sc_att(xl, xr, att, src, dst):
    mesh = plsc.VectorSubcoreMesh(core_axis_name="c", subcore_axis_name="s")
    return pl.kernel(
        _sc_att_body,
        out_type=(
            jax.ShapeDtypeStruct((NC, ACC_ROWS, D), jnp.float32),
            jax.ShapeDtypeStruct((NC, NDG, D), jnp.float32),
        ),
        mesh=mesh,
        scratch_types=[
            pltpu.VMEM((EB,), jnp.int32),
            pltpu.VMEM((EB,), jnp.int32),
            pltpu.VMEM((EB + LANES,), jnp.int32),
            pltpu.VMEM((EB + LANES,), jnp.int32),
            pltpu.VMEM((EB,), jnp.int32),
            pltpu.VMEM((EB,), jnp.int32),
            pltpu.VMEM((EB, D), jnp.float32),
            pltpu.VMEM((EB, D), jnp.float32),
            pltpu.VMEM((EB, D), jnp.float32),
            pltpu.VMEM((EB, D), jnp.float32),
            pltpu.VMEM((EB, D), jnp.float32),
            pltpu.VMEM((EB, D), jnp.float32),
            pltpu.VMEM((D,), jnp.float32),
            pltpu.VMEM_SHARED((ACC_ROWS, D), jnp.float32),
            pltpu.VMEM_SHARED((NDG, D), jnp.float32),
            pltpu.SemaphoreType.DMA,
            pltpu.SemaphoreType.DMA,
        ],
    )(xl, xr, att, src, dst)


_RT = 512
_NRT = NPAD // _RT


def _mm1_body(x_ref, w_ref, o_ref):
    o_ref[0] = jnp.dot(x_ref[...], w_ref[...], preferred_element_type=jnp.float32)


@jax.jit
def _mm1(xp, wcat):
    return pl.pallas_call(
        _mm1_body,
        grid=(2 * HEADS, _NRT),
        in_specs=[
            pl.BlockSpec((_RT, D), lambda j, i: (i, 0)),
            pl.BlockSpec((D, D), lambda j, i: (0, j)),
        ],
        out_specs=pl.BlockSpec((1, _RT, D), lambda j, i: (j, i, 0)),
        out_shape=jax.ShapeDtypeStruct((2 * HEADS, NPAD, D), jnp.float32),
    )(xp, wcat)


_HRT = HALF // _RT


def _norm_head(feat_ref, den_ref):
    num = feat_ref[0]
    den = den_ref[0][:, None]
    return num / (den + 1e-16)


def _mid_body(f0, f1, f2, f3, d0, d1, d2, d3, b1_ref, w_ref, o_ref):
    hs = [_norm_head(f, d) for f, d in ((f0, d0), (f1, d1), (f2, d2), (f3, d3))]
    h = jnp.concatenate(hs, axis=1) + b1_ref[0]
    h = jnp.where(h > 0.0, h, jnp.exp(jnp.minimum(h, 0.0)) - 1.0)
    o_ref[0] = jnp.dot(h, w_ref[...], preferred_element_type=jnp.float32)


@jax.jit
def _mid(p0, p1, p2, p3, b1r, wcat2):
    fspec = pl.BlockSpec((1, _RT, D), lambda j, i: (i // _HRT, i % _HRT, 0))
    dspec = pl.BlockSpec((NC, _RT), lambda j, i: (0, i))
    return pl.pallas_call(
        _mid_body,
        grid=(2, _NRT),
        in_specs=[fspec, fspec, fspec, fspec, dspec, dspec, dspec, dspec,
                  pl.BlockSpec((1, HEADS * D), lambda j, i: (0, 0)),
                  pl.BlockSpec((HEADS * D, D), lambda j, i: (0, j))],
        out_specs=pl.BlockSpec((1, _RT, D), lambda j, i: (j, i, 0)),
        out_shape=jax.ShapeDtypeStruct((2, NPAD, D), jnp.float32),
    )(p0[0], p1[0], p2[0], p3[0],
      p0[1].reshape(NC, NPAD), p1[1].reshape(NC, NPAD),
      p2[1].reshape(NC, NPAD), p3[1].reshape(NC, NPAD), b1r, wcat2)


def _fin_body(f_ref, d_ref, b2_ref, o_ref):
    o_ref[...] = _norm_head(f_ref, d_ref) + b2_ref[0]


@jax.jit
def _fin(q, b2r):
    return pl.pallas_call(
        _fin_body,
        grid=(_NRT,),
        in_specs=[
            pl.BlockSpec((1, _RT, D), lambda i: (i // _HRT, i % _HRT, 0)),
            pl.BlockSpec((NC, _RT), lambda i: (0, i)),
            pl.BlockSpec((1, D), lambda i: (0, 0)),
        ],
        out_specs=pl.BlockSpec((_RT, D), lambda i: (i, 0)),
        out_shape=jax.ShapeDtypeStruct((NPAD, D), jnp.float32),
    )(q[0], q[1].reshape(NC, NPAD), b2r)


def kernel(x, edge_index, W1l, W1r, a1, b1, W2l, W2r, a2, b2):
    xp = jnp.zeros((NPAD, D), jnp.float32).at[:N_NODES].set(x)
    loop = jnp.arange(N_NODES, dtype=jnp.int32)
    pad = jnp.full((EPAD - E_FULL,), N_NODES, dtype=jnp.int32)
    src = jnp.concatenate([edge_index[0].astype(jnp.int32), loop, pad])
    dst = jnp.concatenate([edge_index[1].astype(jnp.int32), loop, pad])

    wcat1 = jnp.concatenate([W1l, W1r], axis=1)
    y1 = _mm1(xp, wcat1)

    parts = [
        _sc_att(y1[h], y1[HEADS + h], a1[h], src, dst) for h in range(HEADS)
    ]

    wcat2 = jnp.concatenate([W2l, W2r], axis=1)
    y2 = _mid(parts[0], parts[1], parts[2], parts[3],
              b1.reshape(1, HEADS * D), wcat2)

    q = _sc_att(y2[0], y2[1], a2[0], src, dst)
    out = _fin(q, b2.reshape(1, D))
    return out[:N_NODES]


# speedup vs baseline: 1.5577x; 1.0194x over previous
import jax
import jax.numpy as jnp
from jax import lax
from jax.experimental import pallas as pl
from jax.experimental.pallas import tpu as pltpu
from jax.experimental.pallas import tpu_sc as plsc

N_NODES = 10000
N_EDGES = 320000
D = 128
HEADS = 4

NPAD = 10240
NC, NS, LANES = 2, 16, 16
E_FULL = N_EDGES + N_NODES
EB = 96
BATCHES = 216
PAIRS = BATCHES // 2
EPT = EB * BATCHES
EPAD = EPT * NS
HALF = NPAD // NC
ACC_ROWS = HALF + 128
ROWS_PER_TILE = ACC_ROWS // NS

NDG = NPAD // D
DG_SLAB = 8
DG_TILES = NDG // DG_SLAB


def _sc_att_body(xl_hbm, xr_hbm, att_hbm, src_hbm, dst_hbm,
                 feat_hbm, den_hbm,
                 src_a, src_b, dst_a, dst_b, den_idx, loc_idx,
                 xl_a, xr_a, xl_b, xr_b,
                 out_rows, den_rows, att_v, shared, den_acc,
                 sem_a, sem_b, sem_c, sem_d):
    c = lax.axis_index("c")
    s = lax.axis_index("s")
    lane = lax.iota(jnp.int32, LANES)

    pltpu.sync_copy(att_hbm, att_v)
    dst_a[pl.ds(EB, LANES)] = jnp.zeros((LANES,), jnp.int32)
    dst_b[pl.ds(EB, LANES)] = jnp.zeros((LANES,), jnp.int32)
    att_c = [att_v[pl.ds(k * LANES, LANES)] for k in range(D // LANES)]
    lanefull = [lane + (k * LANES) for k in range(D // LANES)]

    def _zero_rows(r, carry):
        for kk in range(D // LANES):
            out_rows[r, pl.ds(kk * LANES, LANES)] = jnp.zeros((LANES,), jnp.float32)
        return carry

    lax.fori_loop(0, EB, _zero_rows, 0)

    for off, nr in ((0, EB), (EB, EB), (2 * EB, EB),
                    (3 * EB, ROWS_PER_TILE - 3 * EB)):
        pltpu.sync_copy(
            out_rows.at[pl.ds(0, nr)],
            shared.at[pl.ds(s * ROWS_PER_TILE + off, nr)])

    @pl.when(s < DG_TILES)
    def _zero_den():
        pltpu.sync_copy(out_rows.at[pl.ds(0, DG_SLAB)],
                        den_acc.at[pl.ds(s * DG_SLAB, DG_SLAB)])

    plsc.subcore_barrier()

    def _issue(base, src_v, dst_v, xl_rows, xr_rows, sem):
        pltpu.sync_copy(src_hbm.at[pl.ds(base, EB)], src_v)
        pltpu.sync_copy(dst_hbm.at[pl.ds(base, EB)], dst_v.at[pl.ds(0, EB)])
        pltpu.async_copy(xl_hbm.at[src_v], xl_rows, sem)
        pltpu.async_copy(xr_hbm.at[dst_v.at[pl.ds(0, EB)]], xr_rows, sem)

    def _wait(src_v, dst_v, xl_rows, xr_rows, sem):
        pltpu.make_async_copy(xl_hbm.at[src_v], xl_rows, sem).wait()
        pltpu.make_async_copy(xr_hbm.at[dst_v.at[pl.ds(0, EB)]], xr_rows, sem).wait()

    def _process(dst_v, xl_rows, xr_rows):
        for g in range(EB // LANES):
            dvg = dst_v[pl.ds(g * LANES, LANES)]
            den_idx[pl.ds(g * LANES, LANES)] = lax.shift_right_logical(dvg, 7)
            loc = dvg - c * HALF
            ok = (loc >= 0) & (loc < HALF)
            loc_idx[pl.ds(g * LANES, LANES)] = jnp.where(
                ok, loc, jnp.full((LANES,), HALF, jnp.int32))

        def _edge(e, ecarry):
            xs = []
            acc = jnp.zeros((LANES,), jnp.float32)
            for k in range(D // LANES):
                xlk = xl_rows[e, pl.ds(k * LANES, LANES)]
                xs.append(xlk)
                t = xlk + xr_rows[e, pl.ds(k * LANES, LANES)]
                t = jnp.where(t >= 0.0, t, 0.2 * t)
                acc = acc + t * att_c[k]
            for step in (8, 4, 2, 1):
                acc = acc + acc.at[lane ^ step].get(mode="promise_in_bounds")
            pvec = jnp.exp(acc)
            dv = dst_v[pl.ds(e, LANES)]
            dm = jnp.full((LANES,), dv[0] & (D - 1), jnp.int32)
            for k in range(D // LANES):
                out_rows[e, pl.ds(k * LANES, LANES)] = xs[k] * pvec
                den_rows[e, pl.ds(k * LANES, LANES)] = jnp.where(
                    lanefull[k] == dm, pvec, jnp.zeros((LANES,), jnp.float32))
            return ecarry

        lax.fori_loop(0, EB, _edge, 0)
        ca = pltpu.async_copy(out_rows, shared.at[loc_idx], sem_c, add=True)
        cb = pltpu.async_copy(den_rows, den_acc.at[den_idx], sem_d, add=True)
        ca.wait()
        cb.wait()

    base0 = s * EPT
    _issue(base0, src_a, dst_a, xl_a, xr_a, sem_a)

    def _pair(i, carry):
        base_a = s * EPT + (2 * i) * EB
        base_b = base_a + EB
        base_n = s * EPT + jnp.minimum(2 * i + 2, BATCHES - 1) * EB
        _issue(base_b, src_b, dst_b, xl_b, xr_b, sem_b)
        _wait(src_a, dst_a, xl_a, xr_a, sem_a)
        _process(dst_a, xl_a, xr_a)
        _issue(base_n, src_a, dst_a, xl_a, xr_a, sem_a)
        _wait(src_b, dst_b, xl_b, xr_b, sem_b)
        _process(dst_b, xl_b, xr_b)
        return carry

    lax.fori_loop(0, PAIRS, _pair, 0)
    _wait(src_a, dst_a, xl_a, xr_a, sem_a)
    plsc.subcore_barrier()

    pltpu.sync_copy(shared.at[pl.ds(s * ROWS_PER_TILE, ROWS_PER_TILE)],
                    feat_hbm.at[c, pl.ds(s * ROWS_PER_TILE, ROWS_PER_TILE)])

    @pl.when(s < DG_TILES)
    def _dump_den():
        pltpu.sync_copy(den_acc.at[pl.ds(s * DG_SLAB, DG_SLAB)],
                        den_hbm.at[c, pl.ds(s * DG_SLAB, DG_SLAB)])


@jax.jit
def _sc_att(xl, xr, att, src, dst):
    mesh = plsc.VectorSubcoreMesh(core_axis_name="c", subcore_axis_name="s")
    return pl.kernel(
        _sc_att_body,
        out_type=(
            jax.ShapeDtypeStruct((NC, ACC_ROWS, D), jnp.float32),
            jax.ShapeDtypeStruct((NC, NDG, D), jnp.float32),
        ),
        mesh=mesh,
        scratch_types=[
            pltpu.VMEM((EB,), jnp.int32),
            pltpu.VMEM((EB,), jnp.int32),
            pltpu.VMEM((EB + LANES,), jnp.int32),
            pltpu.VMEM((EB + LANES,), jnp.int32),
            pltpu.VMEM((EB,), jnp.int32),
            pltpu.VMEM((EB,), jnp.int32),
            pltpu.VMEM((EB, D), jnp.float32),
            pltpu.VMEM((EB, D), jnp.float32),
            pltpu.VMEM((EB, D), jnp.float32),
            pltpu.VMEM((EB, D), jnp.float32),
            pltpu.VMEM((EB, D), jnp.float32),
            pltpu.VMEM((EB, D), jnp.float32),
            pltpu.VMEM((D,), jnp.float32),
            pltpu.VMEM_SHARED((ACC_ROWS, D), jnp.float32),
            pltpu.VMEM_SHARED((NDG, D), jnp.float32),
            pltpu.SemaphoreType.DMA,
            pltpu.SemaphoreType.DMA,
            pltpu.SemaphoreType.DMA,
            pltpu.SemaphoreType.DMA,
        ],
    )(xl, xr, att, src, dst)


_RT = 512
_NRT = NPAD // _RT


def _mm1_body(x_ref, w_ref, o_ref):
    o_ref[0] = jnp.dot(x_ref[...], w_ref[...], preferred_element_type=jnp.float32)


@jax.jit
def _mm1(xp, wcat):
    return pl.pallas_call(
        _mm1_body,
        grid=(2 * HEADS, _NRT),
        in_specs=[
            pl.BlockSpec((_RT, D), lambda j, i: (i, 0)),
            pl.BlockSpec((D, D), lambda j, i: (0, j)),
        ],
        out_specs=pl.BlockSpec((1, _RT, D), lambda j, i: (j, i, 0)),
        out_shape=jax.ShapeDtypeStruct((2 * HEADS, NPAD, D), jnp.float32),
    )(xp, wcat)


_HRT = HALF // _RT


def _norm_head(feat_ref, den_ref):
    num = feat_ref[0]
    den = den_ref[0][:, None]
    return num / (den + 1e-16)


def _mid_body(f0, f1, f2, f3, d0, d1, d2, d3, b1_ref, w_ref, o_ref):
    hs = [_norm_head(f, d) for f, d in ((f0, d0), (f1, d1), (f2, d2), (f3, d3))]
    h = jnp.concatenate(hs, axis=1) + b1_ref[0]
    h = jnp.where(h > 0.0, h, jnp.exp(jnp.minimum(h, 0.0)) - 1.0)
    o_ref[0] = jnp.dot(h, w_ref[...], preferred_element_type=jnp.float32)


@jax.jit
def _mid(p0, p1, p2, p3, b1r, wcat2):
    fspec = pl.BlockSpec((1, _RT, D), lambda j, i: (i // _HRT, i % _HRT, 0))
    dspec = pl.BlockSpec((NC, _RT), lambda j, i: (0, i))
    return pl.pallas_call(
        _mid_body,
        grid=(2, _NRT),
        in_specs=[fspec, fspec, fspec, fspec, dspec, dspec, dspec, dspec,
                  pl.BlockSpec((1, HEADS * D), lambda j, i: (0, 0)),
                  pl.BlockSpec((HEADS * D, D), lambda j, i: (0, j))],
        out_specs=pl.BlockSpec((1, _RT, D), lambda j, i: (j, i, 0)),
        out_shape=jax.ShapeDtypeStruct((2, NPAD, D), jnp.float32),
    )(p0[0], p1[0], p2[0], p3[0],
      p0[1].reshape(NC, NPAD), p1[1].reshape(NC, NPAD),
      p2[1].reshape(NC, NPAD), p3[1].reshape(NC, NPAD), b1r, wcat2)


def _fin_body(f_ref, d_ref, b2_ref, o_ref):
    o_ref[...] = _norm_head(f_ref, d_ref) + b2_ref[0]


@jax.jit
def _fin(q, b2r):
    return pl.pallas_call(
        _fin_body,
        grid=(_NRT,),
        in_specs=[
            pl.BlockSpec((1, _RT, D), lambda i: (i // _HRT, i % _HRT, 0)),
            pl.BlockSpec((NC, _RT), lambda i: (0, i)),
            pl.BlockSpec((1, D), lambda i: (0, 0)),
        ],
        out_specs=pl.BlockSpec((_RT, D), lambda i: (i, 0)),
        out_shape=jax.ShapeDtypeStruct((NPAD, D), jnp.float32),
    )(q[0], q[1].reshape(NC, NPAD), b2r)


def kernel(x, edge_index, W1l, W1r, a1, b1, W2l, W2r, a2, b2):
    xp = jnp.zeros((NPAD, D), jnp.float32).at[:N_NODES].set(x)
    loop = jnp.arange(N_NODES, dtype=jnp.int32)
    pad = jnp.full((EPAD - E_FULL,), N_NODES, dtype=jnp.int32)
    src = jnp.concatenate([edge_index[0].astype(jnp.int32), loop, pad])
    dst = jnp.concatenate([edge_index[1].astype(jnp.int32), loop, pad])

    wcat1 = jnp.concatenate([W1l, W1r], axis=1)
    y1 = _mm1(xp, wcat1)

    parts = [
        _sc_att(y1[h], y1[HEADS + h], a1[h], src, dst) for h in range(HEADS)
    ]

    wcat2 = jnp.concatenate([W2l, W2r], axis=1)
    y2 = _mid(parts[0], parts[1], parts[2], parts[3],
              b1.reshape(1, HEADS * D), wcat2)

    q = _sc_att(y2[0], y2[1], a2[0], src, dst)
    out = _fin(q, b2.reshape(1, D))
    return out[:N_NODES]


# speedup vs baseline: 1.5794x; 1.0140x over previous
import jax
import jax.numpy as jnp
from jax import lax
from jax.experimental import pallas as pl
from jax.experimental.pallas import tpu as pltpu
from jax.experimental.pallas import tpu_sc as plsc

N_NODES = 10000
N_EDGES = 320000
D = 128
HEADS = 4

NPAD = 10240
NC, NS, LANES = 2, 16, 16
E_FULL = N_EDGES + N_NODES
EB = 96
BATCHES = 216
PAIRS = BATCHES // 2
EPT = EB * BATCHES
EPAD = EPT * NS
HALF = NPAD // NC
ACC_ROWS = HALF + 128
ROWS_PER_TILE = ACC_ROWS // NS

NDG = NPAD // D
DG_SLAB = 8
DG_TILES = NDG // DG_SLAB


def _make_sc_body(nheads):
    def _sc_att_body(y_hbm, att_hbm, src_hbm, dst_hbm,
                     feat_hbm, den_hbm,
                     src_a, src_b, dst_a, dst_b, den_idx, loc_idx,
                     xl_a, xr_a, xl_b, xr_b,
                     out_rows, den_rows, att_v, shared, den_acc,
                     sem_a, sem_b, sem_c, sem_d):
        c = lax.axis_index("c")
        s = lax.axis_index("s")
        lane = lax.iota(jnp.int32, LANES)

        dst_a[pl.ds(EB, LANES)] = jnp.zeros((LANES,), jnp.int32)
        dst_b[pl.ds(EB, LANES)] = jnp.zeros((LANES,), jnp.int32)
        lanefull = [lane + (k * LANES) for k in range(D // LANES)]

        def _zero_rows(r, carry):
            for kk in range(D // LANES):
                out_rows[r, pl.ds(kk * LANES, LANES)] = jnp.zeros(
                    (LANES,), jnp.float32)
            return carry

        def _zero_shared():
            for off, nr in ((0, EB), (EB, EB), (2 * EB, EB),
                            (3 * EB, ROWS_PER_TILE - 3 * EB)):
                pltpu.sync_copy(
                    out_rows.at[pl.ds(0, nr)],
                    shared.at[pl.ds(s * ROWS_PER_TILE + off, nr)])

            @pl.when(s < DG_TILES)
            def _zero_den():
                pltpu.sync_copy(out_rows.at[pl.ds(0, DG_SLAB)],
                                den_acc.at[pl.ds(s * DG_SLAB, DG_SLAB)])

        def _issue(h, base, src_v, dst_v, xl_rows, xr_rows, sem):
            pltpu.sync_copy(src_hbm.at[pl.ds(base, EB)], src_v)
            pltpu.sync_copy(dst_hbm.at[pl.ds(base, EB)], dst_v.at[pl.ds(0, EB)])
            pltpu.async_copy(y_hbm.at[h].at[src_v], xl_rows, sem)
            pltpu.async_copy(y_hbm.at[nheads + h].at[dst_v.at[pl.ds(0, EB)]],
                             xr_rows, sem)

        def _wait(h, src_v, dst_v, xl_rows, xr_rows, sem):
            pltpu.make_async_copy(y_hbm.at[h].at[src_v], xl_rows, sem).wait()
            pltpu.make_async_copy(y_hbm.at[nheads + h].at[dst_v.at[pl.ds(0, EB)]],
                                  xr_rows, sem).wait()

        def _process(att_c, dst_v, xl_rows, xr_rows):
            for g in range(EB // LANES):
                dvg = dst_v[pl.ds(g * LANES, LANES)]
                den_idx[pl.ds(g * LANES, LANES)] = lax.shift_right_logical(dvg, 7)
                loc = dvg - c * HALF
                ok = (loc >= 0) & (loc < HALF)
                loc_idx[pl.ds(g * LANES, LANES)] = jnp.where(
                    ok, loc, jnp.full((LANES,), HALF, jnp.int32))

            def _edge(e, ecarry):
                xs = []
                acc = jnp.zeros((LANES,), jnp.float32)
                for k in range(D // LANES):
                    xlk = xl_rows[e, pl.ds(k * LANES, LANES)]
                    xs.append(xlk)
                    t = xlk + xr_rows[e, pl.ds(k * LANES, LANES)]
                    t = jnp.where(t >= 0.0, t, 0.2 * t)
                    acc = acc + t * att_c[k]
                for step in (8, 4, 2, 1):
                    acc = acc + acc.at[lane ^ step].get(mode="promise_in_bounds")
                pvec = jnp.exp(acc)
                dv = dst_v[pl.ds(e, LANES)]
                dm = jnp.full((LANES,), dv[0] & (D - 1), jnp.int32)
                for k in range(D // LANES):
                    out_rows[e, pl.ds(k * LANES, LANES)] = xs[k] * pvec
                    den_rows[e, pl.ds(k * LANES, LANES)] = jnp.where(
                        lanefull[k] == dm, pvec,
                        jnp.zeros((LANES,), jnp.float32))
                return ecarry

            lax.fori_loop(0, EB, _edge, 0)
            ca = pltpu.async_copy(out_rows, shared.at[loc_idx], sem_c, add=True)
            cb = pltpu.async_copy(den_rows, den_acc.at[den_idx], sem_d, add=True)
            ca.wait()
            cb.wait()

        lax.fori_loop(0, EB, _zero_rows, 0)
        _zero_shared()
        plsc.subcore_barrier()

        base0 = s * EPT
        for h in range(nheads):
            pltpu.sync_copy(att_hbm.at[h], att_v)
            att_c = [att_v[pl.ds(k * LANES, LANES)] for k in range(D // LANES)]
            _issue(h, base0, src_a, dst_a, xl_a, xr_a, sem_a)

            def _pair(i, carry):
                base_a = s * EPT + (2 * i) * EB
                base_b = base_a + EB
                base_n = s * EPT + jnp.minimum(2 * i + 2, BATCHES - 1) * EB
                _issue(h, base_b, src_b, dst_b, xl_b, xr_b, sem_b)
                _wait(h, src_a, dst_a, xl_a, xr_a, sem_a)
                _process(att_c, dst_a, xl_a, xr_a)
                _issue(h, base_n, src_a, dst_a, xl_a, xr_a, sem_a)
                _wait(h, src_b, dst_b, xl_b, xr_b, sem_b)
                _process(att_c, dst_b, xl_b, xr_b)
                return carry

            lax.fori_loop(0, PAIRS, _pair, 0)
            _wait(h, src_a, dst_a, xl_a, xr_a, sem_a)
            plsc.subcore_barrier()

            pltpu.sync_copy(
                shared.at[pl.ds(s * ROWS_PER_TILE, ROWS_PER_TILE)],
                feat_hbm.at[h, c, pl.ds(s * ROWS_PER_TILE, ROWS_PER_TILE)])

            @pl.when(s < DG_TILES)
            def _dump_den():
                pltpu.sync_copy(den_acc.at[pl.ds(s * DG_SLAB, DG_SLAB)],
                                den_hbm.at[h, c, pl.ds(s * DG_SLAB, DG_SLAB)])

            if h + 1 < nheads:
                lax.fori_loop(0, EB, _zero_rows, 0)
                _zero_shared()
                plsc.subcore_barrier()

    return _sc_att_body


def _sc_att(y_cat, att_mat, src, dst):
    nheads = att_mat.shape[0]
    mesh = plsc.VectorSubcoreMesh(core_axis_name="c", subcore_axis_name="s")
    return pl.kernel(
        _make_sc_body(nheads),
        out_type=(
            jax.ShapeDtypeStruct((nheads, NC, ACC_ROWS, D), jnp.float32),
            jax.ShapeDtypeStruct((nheads, NC, NDG, D), jnp.float32),
        ),
        mesh=mesh,
        scratch_types=[
            pltpu.VMEM((EB,), jnp.int32),
            pltpu.VMEM((EB,), jnp.int32),
            pltpu.VMEM((EB + LANES,), jnp.int32),
            pltpu.VMEM((EB + LANES,), jnp.int32),
            pltpu.VMEM((EB,), jnp.int32),
            pltpu.VMEM((EB,), jnp.int32),
            pltpu.VMEM((EB, D), jnp.float32),
            pltpu.VMEM((EB, D), jnp.float32),
            pltpu.VMEM((EB, D), jnp.float32),
            pltpu.VMEM((EB, D), jnp.float32),
            pltpu.VMEM((EB, D), jnp.float32),
            pltpu.VMEM((EB, D), jnp.float32),
            pltpu.VMEM((D,), jnp.float32),
            pltpu.VMEM_SHARED((ACC_ROWS, D), jnp.float32),
            pltpu.VMEM_SHARED((NDG, D), jnp.float32),
            pltpu.SemaphoreType.DMA,
            pltpu.SemaphoreType.DMA,
            pltpu.SemaphoreType.DMA,
            pltpu.SemaphoreType.DMA,
        ],
    )(y_cat, att_mat, src, dst)


_RT = 512
_NRT = NPAD // _RT


def _mm1_body(x_ref, w_ref, o_ref):
    o_ref[0] = jnp.dot(x_ref[...], w_ref[...], preferred_element_type=jnp.float32)


@jax.jit
def _mm1(xp, wcat):
    return pl.pallas_call(
        _mm1_body,
        grid=(2 * HEADS, _NRT),
        in_specs=[
            pl.BlockSpec((_RT, D), lambda j, i: (i, 0)),
            pl.BlockSpec((D, D), lambda j, i: (0, j)),
        ],
        out_specs=pl.BlockSpec((1, _RT, D), lambda j, i: (j, i, 0)),
        out_shape=jax.ShapeDtypeStruct((2 * HEADS, NPAD, D), jnp.float32),
    )(xp, wcat)


_HRT = HALF // _RT


def _mid_body(f0, f1, f2, f3, d_ref, b1_ref, w_ref, o_ref):
    hs = [f[0, 0] / (d_ref[2 * h][:, None] + 1e-16)
          for h, f in enumerate((f0, f1, f2, f3))]
    h = jnp.concatenate(hs, axis=1) + b1_ref[0]
    h = jnp.where(h > 0.0, h, jnp.exp(jnp.minimum(h, 0.0)) - 1.0)
    o_ref[0] = jnp.dot(h, w_ref[...], preferred_element_type=jnp.float32)


@jax.jit
def _mid(feat, den, b1r, wcat2):
    fs = [pl.BlockSpec((1, 1, _RT, D),
                       lambda j, i, hh=h: (hh, i // _HRT, i % _HRT, 0))
          for h in range(HEADS)]
    return pl.pallas_call(
        _mid_body,
        grid=(2, _NRT),
        in_specs=fs +
        [pl.BlockSpec((HEADS * NC, _RT), lambda j, i: (0, i)),
         pl.BlockSpec((1, HEADS * D), lambda j, i: (0, 0)),
         pl.BlockSpec((HEADS * D, D), lambda j, i: (0, j))],
        out_specs=pl.BlockSpec((1, _RT, D), lambda j, i: (j, i, 0)),
        out_shape=jax.ShapeDtypeStruct((2, NPAD, D), jnp.float32),
    )(feat, feat, feat, feat, den.reshape(HEADS * NC, NPAD), b1r, wcat2)


def _fin_body(f_ref, d_ref, b2_ref, o_ref):
    o_ref[...] = f_ref[0, 0] / (d_ref[0][:, None] + 1e-16) + b2_ref[0]


@jax.jit
def _fin(feat, den, b2r):
    return pl.pallas_call(
        _fin_body,
        grid=(_NRT,),
        in_specs=[
            pl.BlockSpec((1, 1, _RT, D), lambda i: (0, i // _HRT, i % _HRT, 0)),
            pl.BlockSpec((NC, _RT), lambda i: (0, i)),
            pl.BlockSpec((1, D), lambda i: (0, 0)),
        ],
        out_specs=pl.BlockSpec((_RT, D), lambda i: (i, 0)),
        out_shape=jax.ShapeDtypeStruct((NPAD, D), jnp.float32),
    )(feat, den.reshape(NC, NPAD), b2r)


def kernel(x, edge_index, W1l, W1r, a1, b1, W2l, W2r, a2, b2):
    xp = jnp.zeros((NPAD, D), jnp.float32).at[:N_NODES].set(x)
    loop = jnp.arange(N_NODES, dtype=jnp.int32)
    pad = jnp.full((EPAD - E_FULL,), N_NODES, dtype=jnp.int32)
    src = jnp.concatenate([edge_index[0].astype(jnp.int32), loop, pad])
    dst = jnp.concatenate([edge_index[1].astype(jnp.int32), loop, pad])

    wcat1 = jnp.concatenate([W1l, W1r], axis=1)
    y1 = _mm1(xp, wcat1)

    feat1, den1 = _sc_att(y1, a1, src, dst)

    wcat2 = jnp.concatenate([W2l, W2r], axis=1)
    y2 = _mid(feat1, den1, b1.reshape(1, HEADS * D), wcat2)

    feat2, den2 = _sc_att(y2, a2, src, dst)
    out = _fin(feat2, den2, b2.reshape(1, D))
    return out[:N_NODES]


# speedup vs baseline: 1.7959x; 1.1371x over previous
import jax
import jax.numpy as jnp
from jax import lax
from jax.experimental import pallas as pl
from jax.experimental.pallas import tpu as pltpu
from jax.experimental.pallas import tpu_sc as plsc

N_NODES = 10000
N_EDGES = 320000
D = 128
HEADS = 4

NPAD = 10240
NC, NS, LANES = 2, 16, 16
E_FULL = N_EDGES + N_NODES
EB = 64
BATCHES = 324
PAIRS = BATCHES // 2
EPT = EB * BATCHES
EPAD = EPT * NS
HALF = NPAD // NC
ACC_ROWS = HALF + 128
ROWS_PER_TILE = ACC_ROWS // NS

NDG = NPAD // D
DG_SLAB = 8
DG_TILES = NDG // DG_SLAB


def _make_sc_body(nheads):
    def _sc_att_body(y_hbm, att_hbm, src_hbm, dst_hbm,
                     feat_hbm, den_hbm,
                     src_a, src_b, dst_a, dst_b,
                     den_ia, loc_a, den_ib, loc_b,
                     xl_a, xr_a, xl_b, xr_b,
                     out_a, den_a, out_b, den_b, att_v, shared, den_acc,
                     sem_a, sem_b, sem_c, sem_d, sem_e, sem_f):
        c = lax.axis_index("c")
        s = lax.axis_index("s")
        lane = lax.iota(jnp.int32, LANES)

        dst_a[pl.ds(EB, LANES)] = jnp.zeros((LANES,), jnp.int32)
        dst_b[pl.ds(EB, LANES)] = jnp.zeros((LANES,), jnp.int32)
        lanefull = [lane + (k * LANES) for k in range(D // LANES)]

        def _zero_rows(r, carry):
            for kk in range(D // LANES):
                out_a[r, pl.ds(kk * LANES, LANES)] = jnp.zeros(
                    (LANES,), jnp.float32)
            return carry

        def _zero_shared():
            for off, nr in [(o, EB) for o in range(0, 5 * EB, EB)] + \
                           [(5 * EB, ROWS_PER_TILE - 5 * EB)]:
                pltpu.sync_copy(
                    out_a.at[pl.ds(0, nr)],
                    shared.at[pl.ds(s * ROWS_PER_TILE + off, nr)])

            @pl.when(s < DG_TILES)
            def _zero_den():
                pltpu.sync_copy(out_a.at[pl.ds(0, DG_SLAB)],
                                den_acc.at[pl.ds(s * DG_SLAB, DG_SLAB)])

        def _issue(h, base, src_v, dst_v, xl_rows, xr_rows, sem):
            pltpu.sync_copy(src_hbm.at[pl.ds(base, EB)], src_v)
            pltpu.sync_copy(dst_hbm.at[pl.ds(base, EB)], dst_v.at[pl.ds(0, EB)])
            pltpu.async_copy(y_hbm.at[h].at[src_v], xl_rows, sem)
            pltpu.async_copy(y_hbm.at[nheads + h].at[dst_v.at[pl.ds(0, EB)]],
                             xr_rows, sem)

        def _wait(h, src_v, dst_v, xl_rows, xr_rows, sem):
            pltpu.make_async_copy(y_hbm.at[h].at[src_v], xl_rows, sem).wait()
            pltpu.make_async_copy(y_hbm.at[nheads + h].at[dst_v.at[pl.ds(0, EB)]],
                                  xr_rows, sem).wait()

        def _process(first, att_c, dst_v, xl_rows, xr_rows,
                     out_rows, den_rows, loc_idx, den_idx, sem_o, sem_n):
            @pl.when(jnp.logical_not(first))
            def _drain():
                pltpu.make_async_copy(out_rows, shared.at[loc_idx],
                                      sem_o).wait()
                pltpu.make_async_copy(den_rows, den_acc.at[den_idx],
                                      sem_n).wait()

            for g in range(EB // LANES):
                dvg = dst_v[pl.ds(g * LANES, LANES)]
                den_idx[pl.ds(g * LANES, LANES)] = lax.shift_right_logical(dvg, 7)
                loc = dvg - c * HALF
                ok = (loc >= 0) & (loc < HALF)
                loc_idx[pl.ds(g * LANES, LANES)] = jnp.where(
                    ok, loc, jnp.full((LANES,), HALF, jnp.int32))

            def _edge(e, ecarry):
                xs = []
                acc = jnp.zeros((LANES,), jnp.float32)
                for k in range(D // LANES):
                    xlk = xl_rows[e, pl.ds(k * LANES, LANES)]
                    xs.append(xlk)
                    t = xlk + xr_rows[e, pl.ds(k * LANES, LANES)]
                    t = jnp.where(t >= 0.0, t, 0.2 * t)
                    acc = acc + t * att_c[k]
                for step in (8, 4, 2, 1):
                    acc = acc + acc.at[lane ^ step].get(mode="promise_in_bounds")
                pvec = jnp.exp(acc)
                dv = dst_v[pl.ds(e, LANES)]
                dm = jnp.full((LANES,), dv[0] & (D - 1), jnp.int32)
                for k in range(D // LANES):
                    out_rows[e, pl.ds(k * LANES, LANES)] = xs[k] * pvec
                    den_rows[e, pl.ds(k * LANES, LANES)] = jnp.where(
                        lanefull[k] == dm, pvec,
                        jnp.zeros((LANES,), jnp.float32))
                return ecarry

            lax.fori_loop(0, EB, _edge, 0)
            pltpu.async_copy(out_rows, shared.at[loc_idx], sem_o, add=True)
            pltpu.async_copy(den_rows, den_acc.at[den_idx], sem_n, add=True)

        def _drain_scatters():
            pltpu.make_async_copy(out_a, shared.at[loc_a], sem_c).wait()
            pltpu.make_async_copy(den_a, den_acc.at[den_ia], sem_d).wait()
            pltpu.make_async_copy(out_b, shared.at[loc_b], sem_e).wait()
            pltpu.make_async_copy(den_b, den_acc.at[den_ib], sem_f).wait()

        lax.fori_loop(0, EB, _zero_rows, 0)
        _zero_shared()
        plsc.subcore_barrier()

        base0 = s * EPT
        for h in range(nheads):
            pltpu.sync_copy(att_hbm.at[h], att_v)
            att_c = [att_v[pl.ds(k * LANES, LANES)] for k in range(D // LANES)]
            _issue(h, base0, src_a, dst_a, xl_a, xr_a, sem_a)

            def _pair(i, carry):
                base_a = s * EPT + (2 * i) * EB
                base_b = base_a + EB
                base_n = s * EPT + jnp.minimum(2 * i + 2, BATCHES - 1) * EB
                first = i == 0
                _issue(h, base_b, src_b, dst_b, xl_b, xr_b, sem_b)
                _wait(h, src_a, dst_a, xl_a, xr_a, sem_a)
                _process(first, att_c, dst_a, xl_a, xr_a,
                         out_a, den_a, loc_a, den_ia, sem_c, sem_d)
                _issue(h, base_n, src_a, dst_a, xl_a, xr_a, sem_a)
                _wait(h, src_b, dst_b, xl_b, xr_b, sem_b)
                _process(first, att_c, dst_b, xl_b, xr_b,
                         out_b, den_b, loc_b, den_ib, sem_e, sem_f)
                return carry

            lax.fori_loop(0, PAIRS, _pair, 0)
            _wait(h, src_a, dst_a, xl_a, xr_a, sem_a)
            _drain_scatters()
            plsc.subcore_barrier()

            pltpu.sync_copy(
                shared.at[pl.ds(s * ROWS_PER_TILE, ROWS_PER_TILE)],
                feat_hbm.at[h, c, pl.ds(s * ROWS_PER_TILE, ROWS_PER_TILE)])

            @pl.when(s < DG_TILES)
            def _dump_den():
                pltpu.sync_copy(den_acc.at[pl.ds(s * DG_SLAB, DG_SLAB)],
                                den_hbm.at[h, c, pl.ds(s * DG_SLAB, DG_SLAB)])

            if h + 1 < nheads:
                lax.fori_loop(0, EB, _zero_rows, 0)
                _zero_shared()
                plsc.subcore_barrier()

    return _sc_att_body


def _sc_att(y_cat, att_mat, src, dst):
    nheads = att_mat.shape[0]
    mesh = plsc.VectorSubcoreMesh(core_axis_name="c", subcore_axis_name="s")
    return pl.kernel(
        _make_sc_body(nheads),
        out_type=(
            jax.ShapeDtypeStruct((nheads, NC, ACC_ROWS, D), jnp.float32),
            jax.ShapeDtypeStruct((nheads, NC, NDG, D), jnp.float32),
        ),
        mesh=mesh,
        scratch_types=[
            pltpu.VMEM((EB,), jnp.int32),
            pltpu.VMEM((EB,), jnp.int32),
            pltpu.VMEM((EB + LANES,), jnp.int32),
            pltpu.VMEM((EB + LANES,), jnp.int32),
            pltpu.VMEM((EB,), jnp.int32),
            pltpu.VMEM((EB,), jnp.int32),
            pltpu.VMEM((EB,), jnp.int32),
            pltpu.VMEM((EB,), jnp.int32),
            pltpu.VMEM((EB, D), jnp.float32),
            pltpu.VMEM((EB, D), jnp.float32),
            pltpu.VMEM((EB, D), jnp.float32),
            pltpu.VMEM((EB, D), jnp.float32),
            pltpu.VMEM((EB, D), jnp.float32),
            pltpu.VMEM((EB, D), jnp.float32),
            pltpu.VMEM((EB, D), jnp.float32),
            pltpu.VMEM((EB, D), jnp.float32),
            pltpu.VMEM((D,), jnp.float32),
            pltpu.VMEM_SHARED((ACC_ROWS, D), jnp.float32),
            pltpu.VMEM_SHARED((NDG, D), jnp.float32),
            pltpu.SemaphoreType.DMA,
            pltpu.SemaphoreType.DMA,
            pltpu.SemaphoreType.DMA,
            pltpu.SemaphoreType.DMA,
            pltpu.SemaphoreType.DMA,
            pltpu.SemaphoreType.DMA,
        ],
    )(y_cat, att_mat, src, dst)


_RT = 512
_NRT = NPAD // _RT


def _mm1_body(x_ref, w_ref, o_ref):
    o_ref[0] = jnp.dot(x_ref[...], w_ref[...], preferred_element_type=jnp.float32)


@jax.jit
def _mm1(xp, wcat):
    return pl.pallas_call(
        _mm1_body,
        grid=(2 * HEADS, _NRT),
        in_specs=[
            pl.BlockSpec((_RT, D), lambda j, i: (i, 0)),
            pl.BlockSpec((D, D), lambda j, i: (0, j)),
        ],
        out_specs=pl.BlockSpec((1, _RT, D), lambda j, i: (j, i, 0)),
        out_shape=jax.ShapeDtypeStruct((2 * HEADS, NPAD, D), jnp.float32),
    )(xp, wcat)


_HRT = HALF // _RT


def _mid_body(f0, f1, f2, f3, d_ref, b1_ref, w_ref, o_ref):
    hs = [f[0, 0] / (d_ref[2 * h][:, None] + 1e-16)
          for h, f in enumerate((f0, f1, f2, f3))]
    h = jnp.concatenate(hs, axis=1) + b1_ref[0]
    h = jnp.where(h > 0.0, h, jnp.exp(jnp.minimum(h, 0.0)) - 1.0)
    o_ref[0] = jnp.dot(h, w_ref[...], preferred_element_type=jnp.float32)


@jax.jit
def _mid(feat, den, b1r, wcat2):
    fs = [pl.BlockSpec((1, 1, _RT, D),
                       lambda j, i, hh=h: (hh, i // _HRT, i % _HRT, 0))
          for h in range(HEADS)]
    return pl.pallas_call(
        _mid_body,
        grid=(2, _NRT),
        in_specs=fs +
        [pl.BlockSpec((HEADS * NC, _RT), lambda j, i: (0, i)),
         pl.BlockSpec((1, HEADS * D), lambda j, i: (0, 0)),
         pl.BlockSpec((HEADS * D, D), lambda j, i: (0, j))],
        out_specs=pl.BlockSpec((1, _RT, D), lambda j, i: (j, i, 0)),
        out_shape=jax.ShapeDtypeStruct((2, NPAD, D), jnp.float32),
    )(feat, feat, feat, feat, den.reshape(HEADS * NC, NPAD), b1r, wcat2)


def _fin_body(f_ref, d_ref, b2_ref, o_ref):
    o_ref[...] = f_ref[0, 0] / (d_ref[0][:, None] + 1e-16) + b2_ref[0]


@jax.jit
def _fin(feat, den, b2r):
    return pl.pallas_call(
        _fin_body,
        grid=(_NRT,),
        in_specs=[
            pl.BlockSpec((1, 1, _RT, D), lambda i: (0, i // _HRT, i % _HRT, 0)),
            pl.BlockSpec((NC, _RT), lambda i: (0, i)),
            pl.BlockSpec((1, D), lambda i: (0, 0)),
        ],
        out_specs=pl.BlockSpec((_RT, D), lambda i: (i, 0)),
        out_shape=jax.ShapeDtypeStruct((NPAD, D), jnp.float32),
    )(feat, den.reshape(NC, NPAD), b2r)


def kernel(x, edge_index, W1l, W1r, a1, b1, W2l, W2r, a2, b2):
    xp = jnp.zeros((NPAD, D), jnp.float32).at[:N_NODES].set(x)
    loop = jnp.arange(N_NODES, dtype=jnp.int32)
    pad = jnp.full((EPAD - E_FULL,), N_NODES, dtype=jnp.int32)
    src = jnp.concatenate([edge_index[0].astype(jnp.int32), loop, pad])
    dst = jnp.concatenate([edge_index[1].astype(jnp.int32), loop, pad])

    wcat1 = jnp.concatenate([W1l, W1r], axis=1)
    y1 = _mm1(xp, wcat1)

    feat1, den1 = _sc_att(y1, a1, src, dst)

    wcat2 = jnp.concatenate([W2l, W2r], axis=1)
    y2 = _mid(feat1, den1, b1.reshape(1, HEADS * D), wcat2)

    feat2, den2 = _sc_att(y2, a2, src, dst)
    out = _fin(feat2, den2, b2.reshape(1, D))
    return out[:N_NODES]
